# baseline (speedup 1.0000x reference)
import functools
import math

import jax
import jax.numpy as jnp
import numpy as np
from jax import lax
from jax.experimental import pallas as pl
from jax.experimental.pallas import tpu as pltpu
from jax.experimental.pallas import tpu_sc as plsc

D_MODEL = 64
SEQ = 200
BATCH = 4096
HALF = SEQ // 2

NUM_CORES = 2
NUM_SUBCORES = 16
NW = NUM_CORES * NUM_SUBCORES
ROWS = BATCH * SEQ
ROWS_PER_W = ROWS // NW
CHUNKS_PER_W = ROWS_PER_W // HALF


def _pe_table() -> jnp.ndarray:
    pe = np.zeros((SEQ, D_MODEL), dtype=np.float32)
    pos = np.arange(0, SEQ, dtype=np.float32)[:, None]
    k = np.exp(-math.log(10000.0) * np.arange(0, D_MODEL, 2, dtype=np.float32) / D_MODEL)
    pe[:, 0::2] = np.sin(pos * k)
    pe[:, 1::2] = np.cos(pos * k)
    return jnp.asarray(pe)


_MESH = plsc.VectorSubcoreMesh(core_axis_name="c", subcore_axis_name="s")


@functools.partial(
    pl.kernel,
    out_type=jax.ShapeDtypeStruct((ROWS // HALF, HALF, D_MODEL), jnp.float32),
    mesh=_MESH,
    compiler_params=pltpu.CompilerParams(use_tc_tiling_on_sc=False),
    scratch_types=[
        pltpu.VMEM((CHUNKS_PER_W, HALF), jnp.int32),
        pltpu.VMEM((HALF, D_MODEL), jnp.float32),
        pltpu.VMEM((HALF, D_MODEL), jnp.float32),
        pltpu.SemaphoreType.DMA,
        pltpu.SemaphoreType.DMA,
        pltpu.SemaphoreType.DMA,
        pltpu.SemaphoreType.DMA,
    ],
)
def _sc_embed(table_hbm, idx_hbm, pe0_hbm, pe1_hbm, out_hbm,
              idx_v, buf0, buf1, g0, g1, o0, o1):
    wid = lax.axis_index("s") * NUM_CORES + lax.axis_index("c")
    pltpu.sync_copy(idx_hbm.at[pl.ds(wid * CHUNKS_PER_W, CHUNKS_PER_W)], idx_v)

    chunk_base = wid * CHUNKS_PER_W

    def body(g, carry):
        c0 = 2 * g
        pltpu.sync_copy(pe0_hbm, buf0)
        pltpu.async_copy(table_hbm.at[idx_v.at[c0]], buf0, g0, add=True).wait()
        pltpu.async_copy(buf0, out_hbm.at[chunk_base + c0], o0).wait()
        c1 = c0 + 1
        pltpu.sync_copy(pe1_hbm, buf1)
        pltpu.async_copy(table_hbm.at[idx_v.at[c1]], buf1, g1, add=True).wait()
        pltpu.async_copy(buf1, out_hbm.at[chunk_base + c1], o1).wait()
        return carry

    lax.fori_loop(0, CHUNKS_PER_W // 2, body, 0)


def kernel(x, embedding_weight):
    idx = x.astype(jnp.int32).reshape(ROWS // HALF, HALF)
    pe = _pe_table()
    out = _sc_embed(embedding_weight, idx, pe[:HALF], pe[HALF:])
    return out.reshape(BATCH, SEQ, D_MODEL)


# speedup vs baseline: 1.1213x; 1.1213x over previous
import functools
import math

import jax
import jax.numpy as jnp
import numpy as np
from jax import lax
from jax.experimental import pallas as pl
from jax.experimental.pallas import tpu as pltpu
from jax.experimental.pallas import tpu_sc as plsc

D_MODEL = 64
SEQ = 200
BATCH = 4096
HALF = SEQ // 2

NUM_CORES = 2
NUM_SUBCORES = 16
NW = NUM_CORES * NUM_SUBCORES
ROWS = BATCH * SEQ
ROWS_PER_W = ROWS // NW
CHUNKS_PER_W = ROWS_PER_W // HALF
NBUF = 4
ROUNDS = CHUNKS_PER_W // NBUF


def _pe_table() -> jnp.ndarray:
    pe = np.zeros((SEQ, D_MODEL), dtype=np.float32)
    pos = np.arange(0, SEQ, dtype=np.float32)[:, None]
    k = np.exp(-math.log(10000.0) * np.arange(0, D_MODEL, 2, dtype=np.float32) / D_MODEL)
    pe[:, 0::2] = np.sin(pos * k)
    pe[:, 1::2] = np.cos(pos * k)
    return jnp.asarray(pe)


_MESH = plsc.VectorSubcoreMesh(core_axis_name="c", subcore_axis_name="s")


@functools.partial(
    pl.kernel,
    out_type=jax.ShapeDtypeStruct((ROWS // HALF, HALF, D_MODEL), jnp.float32),
    mesh=_MESH,
    compiler_params=pltpu.CompilerParams(use_tc_tiling_on_sc=False),
    scratch_types=[
        pltpu.VMEM((CHUNKS_PER_W, HALF), jnp.int32),
        pltpu.VMEM((NBUF, HALF, D_MODEL), jnp.float32),
        [pltpu.SemaphoreType.DMA] * NBUF,
        [pltpu.SemaphoreType.DMA] * NBUF,
        [pltpu.SemaphoreType.DMA] * NBUF,
    ],
)
def _sc_embed(table_hbm, idx_hbm, pe0_hbm, pe1_hbm, out_hbm,
              idx_v, bufs, psems, gsems, osems):
    wid = lax.axis_index("s") * NUM_CORES + lax.axis_index("c")
    pltpu.sync_copy(idx_hbm.at[pl.ds(wid * CHUNKS_PER_W, CHUNKS_PER_W)], idx_v)

    chunk_base = wid * CHUNKS_PER_W
    pe_src = [pe0_hbm if k % 2 == 0 else pe1_hbm for k in range(NBUF)]

    def one_round(g, wait_out):
        for k in range(NBUF):
            if wait_out:
                pltpu.make_async_copy(
                    bufs.at[k], out_hbm.at[chunk_base], osems[k]).wait()
            pltpu.async_copy(pe_src[k], bufs.at[k], psems[k])
        for k in range(NBUF):
            c = g * NBUF + k
            pltpu.make_async_copy(pe_src[k], bufs.at[k], psems[k]).wait()
            pltpu.async_copy(table_hbm.at[idx_v.at[c]], bufs.at[k], gsems[k],
                             add=True)
        for k in range(NBUF):
            c = g * NBUF + k
            pltpu.make_async_copy(
                table_hbm.at[idx_v.at[c]], bufs.at[k], gsems[k]).wait()
            pltpu.async_copy(bufs.at[k], out_hbm.at[chunk_base + c], osems[k])

    one_round(0, wait_out=False)

    def body(g, carry):
        one_round(g, wait_out=True)
        return carry

    lax.fori_loop(1, ROUNDS, body, 0)

    for k in range(NBUF):
        pltpu.make_async_copy(bufs.at[k], out_hbm.at[chunk_base], osems[k]).wait()


def kernel(x, embedding_weight):
    idx = x.astype(jnp.int32).reshape(ROWS // HALF, HALF)
    pe = _pe_table()
    out = _sc_embed(embedding_weight, idx, pe[:HALF], pe[HALF:])
    return out.reshape(BATCH, SEQ, D_MODEL)
